# baseline (speedup 1.0000x reference)
import functools

import jax
import jax.numpy as jnp
from jax import lax
from jax.experimental import pallas as pl
from jax.experimental.pallas import tpu as pltpu
from jax.experimental.pallas import tpu_sc as plsc

IN_F = 16
OUT_F = 32
NUM_PT = 256
K = 16
UP_FTR = 2
B = 8
N = NUM_PT * UP_FTR
D_IN = NUM_PT * IN_F
D_OUT = NUM_PT * UP_FTR * OUT_F

MLP_BLK = 512
L = 16
NTILES = 32
ROWS_PER_TILE = (B * N) // NTILES
TILES_PER_BATCH = N // ROWS_PER_TILE


def _mlp_kernel(x_ref, w_ref, b_ref, out_ref):
    acc = lax.dot_general(
        x_ref[...].astype(jnp.bfloat16), w_ref[...].astype(jnp.bfloat16),
        dimension_numbers=(((1,), (1,)), ((), ())),
        preferred_element_type=jnp.float32,
    )
    out_ref[...] = acc + b_ref[...]


def _prep_kernel(feat_ref, wt_ref, bg_ref, dist_ref, sup_ref):
    feat = feat_ref[0, :, :]
    g = lax.dot_general(
        feat, feat, dimension_numbers=(((0,), (0,)), ((), ())),
        preferred_element_type=jnp.float32,
        precision=lax.Precision.HIGHEST,
    )
    sq = feat * feat
    n_row = jnp.sum(sq, axis=0, keepdims=True)
    ones = jnp.ones((OUT_F, 1), dtype=jnp.float32)
    n_col = lax.dot_general(
        sq, ones, dimension_numbers=(((0,), (0,)), ((), ())),
        preferred_element_type=jnp.float32,
        precision=lax.Precision.HIGHEST,
    )
    dist_ref[0, :, :] = n_col + n_row - 2.0 * g

    support_t = lax.dot_general(
        wt_ref[...], feat, dimension_numbers=(((1,), (0,)), ((), ())),
        preferred_element_type=jnp.float32,
        precision=lax.Precision.HIGHEST,
    )
    sup_ref[0, :, :] = support_t + bg_ref[...]


def _merge16(ak, av, bk, bv):
    rk = lax.rev(bk, (0,))
    rv = lax.rev(bv, (0,))
    m = ak <= rk
    ck = jnp.where(m, ak, rk)
    cv = jnp.where(m, av, rv)
    return plsc.sort_key_val(ck, cv)


def _sc_body(d_hbm, s_hbm, out_hbm, d_v, s_v, ob_v):
    wid = lax.axis_index("s") * 2 + lax.axis_index("c")
    b = wid // TILES_PER_BATCH
    q0 = (wid % TILES_PER_BATCH) * ROWS_PER_TILE
    pltpu.sync_copy(d_hbm.at[pl.ds(wid * ROWS_PER_TILE, ROWS_PER_TILE)], d_v)
    pltpu.sync_copy(s_hbm.at[b], s_v)

    iota = lax.iota(jnp.int32, L)

    def row_body(j, _):
        pairs = []
        for c in range(N // L):
            keys = d_v[j, pl.ds(c * L, L)]
            vals = iota + (c * L)
            pairs.append(plsc.sort_key_val(keys, vals))
        while len(pairs) > 1:
            nxt = []
            for i in range(0, len(pairs), 2):
                nxt.append(_merge16(pairs[i][0], pairs[i][1],
                                    pairs[i + 1][0], pairs[i + 1][1]))
            pairs = nxt
        top_idx = pairs[0][1]

        acc0 = jnp.zeros((L,), jnp.float32)
        acc1 = jnp.zeros((L,), jnp.float32)
        for k in range(K):
            sel = jnp.sum(jnp.where(iota == k, top_idx, 0))
            addr0 = sel + N * iota
            acc0 = acc0 + plsc.load_gather(s_v, [addr0])
            acc1 = acc1 + plsc.load_gather(s_v, [addr0 + L * N])
        acc0 = acc0 * (1.0 / K)
        acc1 = acc1 * (1.0 / K)

        jcol = jnp.full((L,), 0, jnp.int32) + j
        plsc.store_scatter(ob_v, [iota, jcol], acc0)
        plsc.store_scatter(ob_v, [iota + L, jcol], acc1)
        return 0

    lax.fori_loop(0, ROWS_PER_TILE, row_body, 0)
    pltpu.sync_copy(ob_v, out_hbm.at[b, :, pl.ds(q0, ROWS_PER_TILE)])


@jax.jit
def kernel(input, W_mlp, b_mlp, W_gcn, b_gcn):
    x = input.astype(jnp.float32)
    num_blk = D_OUT // MLP_BLK
    b2 = b_mlp.reshape(1, D_OUT)
    feat_flat = pl.pallas_call(
        _mlp_kernel,
        grid=(num_blk,),
        in_specs=[
            pl.BlockSpec((B, D_IN), lambda j: (0, 0)),
            pl.BlockSpec((MLP_BLK, D_IN), lambda j: (j, 0)),
            pl.BlockSpec((1, MLP_BLK), lambda j: (0, j)),
        ],
        out_specs=pl.BlockSpec((B, MLP_BLK), lambda j: (0, j)),
        out_shape=jax.ShapeDtypeStruct((B, D_OUT), jnp.float32),
    )(x, W_mlp, b2)
    feat = feat_flat.reshape(B, OUT_F, N)

    wt = W_gcn[0].T
    bg = b_gcn[0]
    dist, sup = pl.pallas_call(
        _prep_kernel,
        grid=(B,),
        in_specs=[
            pl.BlockSpec((1, OUT_F, N), lambda b: (b, 0, 0)),
            pl.BlockSpec((OUT_F, OUT_F), lambda b: (0, 0)),
            pl.BlockSpec((OUT_F, 1), lambda b: (0, 0)),
        ],
        out_specs=[
            pl.BlockSpec((1, N, N), lambda b: (b, 0, 0)),
            pl.BlockSpec((1, OUT_F, N), lambda b: (b, 0, 0)),
        ],
        out_shape=[
            jax.ShapeDtypeStruct((B, N, N), jnp.float32),
            jax.ShapeDtypeStruct((B, OUT_F, N), jnp.float32),
        ],
    )(feat, wt, bg)

    sc_call = functools.partial(
        pl.kernel,
        mesh=plsc.VectorSubcoreMesh(core_axis_name="c", subcore_axis_name="s"),
        out_type=jax.ShapeDtypeStruct((B, OUT_F, N), jnp.float32),
        scratch_types=[
            pltpu.VMEM((ROWS_PER_TILE, N), jnp.float32),
            pltpu.VMEM((OUT_F * N,), jnp.float32),
            pltpu.VMEM((OUT_F, ROWS_PER_TILE), jnp.float32),
        ],
        compiler_params=pltpu.CompilerParams(needs_layout_passes=False),
    )(_sc_body)
    out = sc_call(dist.reshape(B * N, N), sup.reshape(B, OUT_F * N))

    return out.reshape(B, D_OUT)


# speedup vs baseline: 1.0253x; 1.0253x over previous
import functools

import jax
import jax.numpy as jnp
from jax import lax
from jax.experimental import pallas as pl
from jax.experimental.pallas import tpu as pltpu
from jax.experimental.pallas import tpu_sc as plsc

IN_F = 16
OUT_F = 32
NUM_PT = 256
K = 16
UP_FTR = 2
B = 8
N = NUM_PT * UP_FTR
D_IN = NUM_PT * IN_F
D_OUT = NUM_PT * UP_FTR * OUT_F

MLP_BLK = 512
L = 16
NTILES = 32
ROWS_PER_TILE = (B * N) // NTILES
TILES_PER_BATCH = N // ROWS_PER_TILE


def _mlp_kernel(x_ref, w_ref, b_ref, out_ref):
    acc = lax.dot_general(
        x_ref[...].astype(jnp.bfloat16), w_ref[...].astype(jnp.bfloat16),
        dimension_numbers=(((1,), (1,)), ((), ())),
        preferred_element_type=jnp.float32,
    )
    out_ref[...] = acc + b_ref[...]


def _prep_kernel(feat_ref, wt_ref, bg_ref, dist_ref, sup_ref):
    feat = feat_ref[0, :, :]
    g = lax.dot_general(
        feat, feat, dimension_numbers=(((0,), (0,)), ((), ())),
        preferred_element_type=jnp.float32,
        precision=lax.Precision.HIGHEST,
    )
    sq = feat * feat
    n_row = jnp.sum(sq, axis=0, keepdims=True)
    ones = jnp.ones((OUT_F, 1), dtype=jnp.float32)
    n_col = lax.dot_general(
        sq, ones, dimension_numbers=(((0,), (0,)), ((), ())),
        preferred_element_type=jnp.float32,
        precision=lax.Precision.HIGHEST,
    )
    dist_ref[0, :, :] = n_col + n_row - 2.0 * g

    support_t = lax.dot_general(
        wt_ref[...], feat, dimension_numbers=(((1,), (0,)), ((), ())),
        preferred_element_type=jnp.float32,
        precision=lax.Precision.HIGHEST,
    )
    sup_ref[0, :, :] = support_t + bg_ref[...]


def _merge16(ak, av, bk, bv, descending=None):
    m = ak <= bk
    ck = jnp.where(m, ak, bk)
    cv = jnp.where(m, av, bv)
    if descending is None:
        return ck, cv
    return plsc.sort_key_val(ck, cv, descending=descending)


def _sc_body(d_hbm, s_hbm, out_hbm, d_v, s_v, ob_v):
    wid = lax.axis_index("s") * 2 + lax.axis_index("c")
    b = wid // TILES_PER_BATCH
    q0 = (wid % TILES_PER_BATCH) * ROWS_PER_TILE
    pltpu.sync_copy(d_hbm.at[pl.ds(wid * ROWS_PER_TILE, ROWS_PER_TILE)], d_v)
    pltpu.sync_copy(s_hbm.at[b], s_v)

    iota = lax.iota(jnp.int32, L)

    nxi = N * iota

    def row_body(j, _):
        pairs = []
        for c in range(N // L):
            keys = d_v[j, pl.ds(c * L, L)]
            vals = iota + (c * L)
            pairs.append(plsc.sort_key_val(keys, vals, descending=bool(c % 2)))
        while len(pairs) > 2:
            nxt = []
            for i in range(0, len(pairs), 2):
                nxt.append(_merge16(pairs[i][0], pairs[i][1],
                                    pairs[i + 1][0], pairs[i + 1][1],
                                    descending=bool((i // 2) % 2)))
            pairs = nxt
        _, top_idx = _merge16(pairs[0][0], pairs[0][1],
                              pairs[1][0], pairs[1][1])

        acc0 = jnp.zeros((L,), jnp.float32)
        acc1 = jnp.zeros((L,), jnp.float32)
        for k in range(K):
            kvec = jnp.full((L,), k, jnp.int32)
            sel = jnp.take_along_axis(top_idx, kvec, axis=0)
            addr0 = sel + nxi
            acc0 = acc0 + plsc.load_gather(s_v, [addr0])
            acc1 = acc1 + plsc.load_gather(s_v, [addr0 + L * N])
        acc0 = acc0 * (1.0 / K)
        acc1 = acc1 * (1.0 / K)

        jcol = jnp.full((L,), 0, jnp.int32) + j
        plsc.store_scatter(ob_v, [iota, jcol], acc0)
        plsc.store_scatter(ob_v, [iota + L, jcol], acc1)
        return 0

    lax.fori_loop(0, ROWS_PER_TILE, row_body, 0)
    pltpu.sync_copy(ob_v, out_hbm.at[b, :, pl.ds(q0, ROWS_PER_TILE)])


@jax.jit
def kernel(input, W_mlp, b_mlp, W_gcn, b_gcn):
    x = input.astype(jnp.float32)
    num_blk = D_OUT // MLP_BLK
    b2 = b_mlp.reshape(1, D_OUT)
    feat_flat = pl.pallas_call(
        _mlp_kernel,
        grid=(num_blk,),
        in_specs=[
            pl.BlockSpec((B, D_IN), lambda j: (0, 0)),
            pl.BlockSpec((MLP_BLK, D_IN), lambda j: (j, 0)),
            pl.BlockSpec((1, MLP_BLK), lambda j: (0, j)),
        ],
        out_specs=pl.BlockSpec((B, MLP_BLK), lambda j: (0, j)),
        out_shape=jax.ShapeDtypeStruct((B, D_OUT), jnp.float32),
    )(x, W_mlp, b2)
    feat = feat_flat.reshape(B, OUT_F, N)

    wt = W_gcn[0].T
    bg = b_gcn[0]
    dist, sup = pl.pallas_call(
        _prep_kernel,
        grid=(B,),
        in_specs=[
            pl.BlockSpec((1, OUT_F, N), lambda b: (b, 0, 0)),
            pl.BlockSpec((OUT_F, OUT_F), lambda b: (0, 0)),
            pl.BlockSpec((OUT_F, 1), lambda b: (0, 0)),
        ],
        out_specs=[
            pl.BlockSpec((1, N, N), lambda b: (b, 0, 0)),
            pl.BlockSpec((1, OUT_F, N), lambda b: (b, 0, 0)),
        ],
        out_shape=[
            jax.ShapeDtypeStruct((B, N, N), jnp.float32),
            jax.ShapeDtypeStruct((B, OUT_F, N), jnp.float32),
        ],
    )(feat, wt, bg)

    sc_call = functools.partial(
        pl.kernel,
        mesh=plsc.VectorSubcoreMesh(core_axis_name="c", subcore_axis_name="s"),
        out_type=jax.ShapeDtypeStruct((B, OUT_F, N), jnp.float32),
        scratch_types=[
            pltpu.VMEM((ROWS_PER_TILE, N), jnp.float32),
            pltpu.VMEM((OUT_F * N,), jnp.float32),
            pltpu.VMEM((OUT_F, ROWS_PER_TILE), jnp.float32),
        ],
        compiler_params=pltpu.CompilerParams(needs_layout_passes=False),
    )(_sc_body)
    out = sc_call(dist.reshape(B * N, N), sup.reshape(B, OUT_F * N))

    return out.reshape(B, D_OUT)
